# Initial kernel scaffold; baseline (speedup 1.0000x reference)
import jax
import jax.numpy as jnp
from jax.experimental import pallas as pl


def kernel(table, seq_len):
    raise NotImplementedError("write your pallas kernel here")


# speedup vs baseline: 1.0739x; 1.0739x over previous
import functools

import jax
import jax.numpy as jnp
from jax import lax
from jax.experimental import pallas as pl
from jax.experimental.pallas import tpu as pltpu
from jax.experimental.pallas import tpu_sc as plsc

_NUM_HEADS = 12
_HEAD_DIM = 64
_MAX_REL = 128
_SEQ = 512
_VOCAB = 2 * _MAX_REL + 1
_PAD_L = _SEQ - 1 - _MAX_REL
_PW = 1024
_LANES = 16
_NSHIFT = 8


def _sc_rel_pos(table):
    info = plsc.get_sparse_core_info()
    num_cores = info.num_cores
    num_subcores = info.num_subcores
    nw = num_cores * num_subcores
    pairs = _NUM_HEADS * _SEQ
    per = pairs // nw
    assert per * nw == pairs
    p8_rows = _NSHIFT * _HEAD_DIM
    rows_per_tile = p8_rows // num_subcores

    mesh = plsc.VectorSubcoreMesh(core_axis_name="c", subcore_axis_name="s")

    @functools.partial(
        pl.kernel,
        mesh=mesh,
        out_type=jax.ShapeDtypeStruct(
            (_NUM_HEADS, _HEAD_DIM, _SEQ, _SEQ), jnp.float32),
        scratch_types=[
            pltpu.VMEM((_VOCAB, _HEAD_DIM), jnp.float32),
            pltpu.VMEM((rows_per_tile, _PW), jnp.float32),
            pltpu.VMEM_SHARED((p8_rows, _PW), jnp.float32),
            pltpu.SemaphoreType.DMA,
            pltpu.SemaphoreType.DMA,
        ],
        compiler_params=pltpu.CompilerParams(use_tc_tiling_on_sc=False, needs_layout_passes=False),
    )
    def k(table_hbm, out_hbm, table_v, build_v, p8_s, sem_in, sem_out):
        sid = lax.axis_index("s")
        wid = sid * num_cores + lax.axis_index("c")

        pltpu.async_copy(table_hbm, table_v, sem_in).wait()

        lane = lax.iota(jnp.int32, _LANES)

        def build_row(rr, carry):
            rd = sid * rows_per_tile + rr
            r = rd // _HEAD_DIM
            d = rd - r * _HEAD_DIM
            dv = jnp.full((_LANES,), d, jnp.int32)

            def build_chunk(c, inner):
                q = c * _LANES + lane
                pos = jnp.clip(q + r - _PAD_L, 0, _VOCAB - 1)
                vals = plsc.load_gather(table_v, [pos, dv])
                build_v[rr, pl.ds(c * _LANES, _LANES)] = vals
                return inner

            return lax.fori_loop(0, _PW // _LANES, build_chunk, carry)

        lax.fori_loop(0, rows_per_tile, build_row, 0)

        pltpu.sync_copy(
            build_v, p8_s.at[pl.ds(sid * rows_per_tile, rows_per_tile), :])
        plsc.subcore_barrier()

        def fire(t, carry):
            pair = wid * per + t
            h = pair // _SEQ
            i = pair - h * _SEQ
            start = (_SEQ - 1) - i
            r = lax.rem(start, _NSHIFT)
            astart = pl.multiple_of(start - r, _NSHIFT)
            src = p8_s.at[pl.ds(r * _HEAD_DIM, _HEAD_DIM),
                          pl.ds(astart, _SEQ)]
            dst = out_hbm.at[h, :, i, :]
            pltpu.make_async_copy(src, dst, sem_out).start()
            return carry

        lax.fori_loop(0, per, fire, 0)

        def drain(t, carry):
            pltpu.make_async_copy(
                p8_s.at[pl.ds(0, _HEAD_DIM), pl.ds(0, _SEQ)],
                out_hbm.at[0, :, 0, :],
                sem_out,
            ).wait()
            return carry

        lax.fori_loop(0, per, drain, 0)

    return k(table)


def kernel(table, seq_len):
    del seq_len
    return _sc_rel_pos(table)


# speedup vs baseline: 1.2283x; 1.1437x over previous
import functools

import jax
import jax.numpy as jnp
from jax import lax
from jax.experimental import pallas as pl
from jax.experimental.pallas import tpu as pltpu
from jax.experimental.pallas import tpu_sc as plsc

_NUM_HEADS = 12
_HEAD_DIM = 64
_MAX_REL = 128
_SEQ = 512
_VOCAB = 2 * _MAX_REL + 1
_PAD_L = _SEQ - 1 - _MAX_REL
_PW = 1024
_LANES = 16
_NSHIFT = 8


def _sc_rel_pos(table):
    info = plsc.get_sparse_core_info()
    num_cores = info.num_cores
    num_subcores = info.num_subcores
    nw = num_cores * num_subcores
    pairs = _NUM_HEADS * _SEQ
    per = pairs // nw
    assert per * nw == pairs
    p8_rows = _NSHIFT * _HEAD_DIM
    rows_per_tile = p8_rows // num_subcores

    mesh = plsc.VectorSubcoreMesh(core_axis_name="c", subcore_axis_name="s")

    @functools.partial(
        pl.kernel,
        mesh=mesh,
        out_type=jax.ShapeDtypeStruct(
            (_NUM_HEADS, _HEAD_DIM, _SEQ, _SEQ), jnp.float32),
        scratch_types=[
            pltpu.VMEM((_VOCAB, _HEAD_DIM), jnp.float32),
            pltpu.VMEM((_HEAD_DIM, _PW), jnp.float32),
            pltpu.VMEM_SHARED((p8_rows, _PW), jnp.float32),
            pltpu.SemaphoreType.DMA,
            pltpu.SemaphoreType.DMA,
        ],
        compiler_params=pltpu.CompilerParams(use_tc_tiling_on_sc=False, needs_layout_passes=False),
    )
    def k(table_hbm, out_hbm, table_v, pr_v, p8_s, sem_in, sem_out):
        sid = lax.axis_index("s")
        wid = sid * num_cores + lax.axis_index("c")

        pltpu.async_copy(table_hbm, table_v, sem_in).wait()

        lane = lax.iota(jnp.int32, _LANES)

        def build_row(rr, carry):
            rd = sid * rows_per_tile + rr
            r = rd // _HEAD_DIM
            d = rd - r * _HEAD_DIM
            dv = jnp.full((_LANES,), d, jnp.int32)

            def build_chunk(c, inner):
                q = c * _LANES + lane
                pos = jnp.clip(q + r - _PAD_L, 0, _VOCAB - 1)
                vals = plsc.load_gather(table_v, [pos, dv])
                pr_v[rr, pl.ds(c * _LANES, _LANES)] = vals
                return inner

            return lax.fori_loop(0, _PW // _LANES, build_chunk, carry)

        lax.fori_loop(0, rows_per_tile, build_row, 0)

        pltpu.sync_copy(
            pr_v.at[pl.ds(0, rows_per_tile), :],
            p8_s.at[pl.ds(sid * rows_per_tile, rows_per_tile), :])
        plsc.subcore_barrier()

        r_w = lax.rem(wid, _NSHIFT)
        kk = wid // _NSHIFT
        pltpu.sync_copy(
            p8_s.at[pl.ds(r_w * _HEAD_DIM, _HEAD_DIM), :], pr_v)

        per_class = pairs // _NSHIFT
        i_per_class = _SEQ // _NSHIFT

        def fire(t, carry):
            g = kk * per + t
            h = g // i_per_class
            m = g - h * i_per_class
            i = m * _NSHIFT + (_NSHIFT - 1) - r_w
            astart = pl.multiple_of((_SEQ - _NSHIFT) - m * _NSHIFT, _NSHIFT)
            src = pr_v.at[:, pl.ds(astart, _SEQ)]
            dst = out_hbm.at[h, :, i, :]
            pltpu.make_async_copy(src, dst, sem_out).start()
            return carry

        lax.fori_loop(0, per, fire, 0)

        def drain(t, carry):
            pltpu.make_async_copy(
                pr_v.at[:, pl.ds(0, _SEQ)],
                out_hbm.at[0, :, 0, :],
                sem_out,
            ).wait()
            return carry

        lax.fori_loop(0, per, drain, 0)

    return k(table)


def kernel(table, seq_len):
    del seq_len
    return _sc_rel_pos(table)
